# baseline (speedup 1.0000x reference)
import functools

import jax
import jax.numpy as jnp
from jax import lax
from jax.experimental import pallas as pl
from jax.experimental.pallas import tpu as pltpu
from jax.experimental.pallas import tpu_sc as plsc

B = 16384
D = 128
NC, NS = 2, 16
NW = NC * NS
NCH = 2
CH = B // NCH
B_PER_W = CH // NW
HALF = B_PER_W // 2


def _gather_body(user_tab, item_tab, uidx_hbm, iidx_hbm, ue_hbm, ie_hbm,
                 uidx_v, iidx_v, r0, r1, g0, g1, w0, w1):
    wid = lax.axis_index("s") * NC + lax.axis_index("c")
    base = wid * B_PER_W
    pltpu.sync_copy(uidx_hbm.at[pl.ds(base, B_PER_W)], uidx_v)
    pltpu.sync_copy(iidx_hbm.at[pl.ds(base, B_PER_W)], iidx_v)

    items = [
        (uidx_v, user_tab, ue_hbm, 0),
        (uidx_v, user_tab, ue_hbm, HALF),
        (iidx_v, item_tab, ie_hbm, 0),
        (iidx_v, item_tab, ie_hbm, HALF),
    ]
    bufs = (r0, r1)
    gsems = (g0, g1)
    wsems = (w0, w1)

    copies = [None, None, None, None]
    writes = [None, None]
    for k, (idx_v, tab, out_hbm, off) in enumerate(items):
        b = k % 2
        if writes[b] is not None:
            writes[b].wait()
        copies[k] = pltpu.async_copy(
            tab.at[idx_v.at[pl.ds(off, HALF)]], bufs[b], gsems[b])
        if k >= 1:
            copies[k - 1].wait()
            pk = k - 1
            pidx, ptab, pout, poff = items[pk]
            writes[pk % 2] = pltpu.async_copy(
                bufs[pk % 2], pout.at[pl.ds(base + poff, HALF)], wsems[pk % 2])
    copies[3].wait()
    writes[0].wait()
    writes[1] = pltpu.async_copy(bufs[1], ie_hbm.at[pl.ds(base + HALF, HALF)],
                                 wsems[1])
    writes[1].wait()


@functools.lru_cache(maxsize=1)
def _gather_kernel():
    mesh = plsc.VectorSubcoreMesh(core_axis_name="c", subcore_axis_name="s",
                                  num_cores=NC, num_subcores=NS)
    return pl.kernel(
        _gather_body,
        out_type=[
            jax.ShapeDtypeStruct((CH, D), jnp.float32),
            jax.ShapeDtypeStruct((CH, D), jnp.float32),
        ],
        mesh=mesh,
        scratch_types=[
            pltpu.VMEM((B_PER_W,), jnp.int32),
            pltpu.VMEM((B_PER_W,), jnp.int32),
            pltpu.VMEM((HALF, D), jnp.float32),
            pltpu.VMEM((HALF, D), jnp.float32),
            pltpu.SemaphoreType.DMA,
            pltpu.SemaphoreType.DMA,
            pltpu.SemaphoreType.DMA,
            pltpu.SemaphoreType.DMA,
        ],
    )


def _mlp_body(ue_ref, ie_ref, w1u_ref, w1i_ref, b1_ref, w2_ref, b2_ref,
              w3t_ref, b3_ref, out_ref):
    h = jnp.dot(ue_ref[...], w1u_ref[...], preferred_element_type=jnp.float32)
    h += jnp.dot(ie_ref[...], w1i_ref[...], preferred_element_type=jnp.float32)
    h = jnp.maximum(h + b1_ref[...], 0.0)
    h = jnp.dot(h, w2_ref[...], preferred_element_type=jnp.float32)
    h = jnp.maximum(h + b2_ref[...], 0.0)
    o = jax.lax.dot_general(w3t_ref[...], h, (((1,), (1,)), ((), ())),
                            preferred_element_type=jnp.float32)
    out_ref[...] = jax.nn.sigmoid(o + b3_ref[...])[None]


_BB = 2048


def _mlp(ue, ie, w1u, w1i, b1, w2, b2, w3t, b3):
    return pl.pallas_call(
        _mlp_body,
        grid=(CH // _BB,),
        in_specs=[
            pl.BlockSpec((_BB, D), lambda i: (i, 0)),
            pl.BlockSpec((_BB, D), lambda i: (i, 0)),
            pl.BlockSpec((D, 64), lambda i: (0, 0)),
            pl.BlockSpec((D, 64), lambda i: (0, 0)),
            pl.BlockSpec((1, 64), lambda i: (0, 0)),
            pl.BlockSpec((64, 32), lambda i: (0, 0)),
            pl.BlockSpec((1, 32), lambda i: (0, 0)),
            pl.BlockSpec((1, 32), lambda i: (0, 0)),
            pl.BlockSpec((1, 1), lambda i: (0, 0)),
        ],
        out_specs=pl.BlockSpec((1, 1, _BB), lambda i: (i, 0, 0)),
        out_shape=jax.ShapeDtypeStruct((CH // _BB, 1, _BB), jnp.float32),
    )(ue, ie, w1u, w1i, b1, w2, b2, w3t, b3)


@jax.jit
def kernel(user, item, user_table, item_table, W1, b1, W2, b2, W3, b3):
    w1u, w1i = W1[:D], W1[D:]
    b1r = b1.reshape(1, 64)
    b2r = b2.reshape(1, 32)
    w3t = W3.reshape(1, 32)
    b3r = b3.reshape(1, 1)
    gather = _gather_kernel()
    outs = []
    for c in range(NCH):
        ue, ie = gather(user_table, item_table,
                        user[c * CH:(c + 1) * CH],
                        item[c * CH:(c + 1) * CH])
        outs.append(_mlp(ue, ie, w1u, w1i, b1r, W2, b2r, w3t, b3r))
    return jnp.concatenate(outs, axis=0).reshape(B)


# speedup vs baseline: 1.1019x; 1.1019x over previous
import functools

import jax
import jax.numpy as jnp
from jax import lax
from jax.experimental import pallas as pl
from jax.experimental.pallas import tpu as pltpu
from jax.experimental.pallas import tpu_sc as plsc

B = 16384
D = 128
NC, NS = 2, 16
NW = NC * NS
NCH = 1
CH = B // NCH
B_PER_W = CH // NW
HALF = B_PER_W // 2


def _gather_body(user_tab, item_tab, uidx_hbm, iidx_hbm, ue_hbm, ie_hbm,
                 uidx_v, iidx_v, r0, r1, g0, g1, w0, w1):
    wid = lax.axis_index("s") * NC + lax.axis_index("c")
    base = wid * B_PER_W
    pltpu.sync_copy(uidx_hbm.at[pl.ds(base, B_PER_W)], uidx_v)
    pltpu.sync_copy(iidx_hbm.at[pl.ds(base, B_PER_W)], iidx_v)

    items = [
        (uidx_v, user_tab, ue_hbm, 0),
        (uidx_v, user_tab, ue_hbm, HALF),
        (iidx_v, item_tab, ie_hbm, 0),
        (iidx_v, item_tab, ie_hbm, HALF),
    ]
    bufs = (r0, r1)
    gsems = (g0, g1)
    wsems = (w0, w1)

    copies = [None, None, None, None]
    writes = [None, None]
    for k, (idx_v, tab, out_hbm, off) in enumerate(items):
        b = k % 2
        if writes[b] is not None:
            writes[b].wait()
        copies[k] = pltpu.async_copy(
            tab.at[idx_v.at[pl.ds(off, HALF)]], bufs[b], gsems[b])
        if k >= 1:
            copies[k - 1].wait()
            pk = k - 1
            pidx, ptab, pout, poff = items[pk]
            writes[pk % 2] = pltpu.async_copy(
                bufs[pk % 2], pout.at[pl.ds(base + poff, HALF)], wsems[pk % 2])
    copies[3].wait()
    writes[0].wait()
    writes[1] = pltpu.async_copy(bufs[1], ie_hbm.at[pl.ds(base + HALF, HALF)],
                                 wsems[1])
    writes[1].wait()


@functools.lru_cache(maxsize=1)
def _gather_kernel():
    mesh = plsc.VectorSubcoreMesh(core_axis_name="c", subcore_axis_name="s",
                                  num_cores=NC, num_subcores=NS)
    return pl.kernel(
        _gather_body,
        out_type=[
            jax.ShapeDtypeStruct((CH, D), jnp.float32),
            jax.ShapeDtypeStruct((CH, D), jnp.float32),
        ],
        mesh=mesh,
        scratch_types=[
            pltpu.VMEM((B_PER_W,), jnp.int32),
            pltpu.VMEM((B_PER_W,), jnp.int32),
            pltpu.VMEM((HALF, D), jnp.float32),
            pltpu.VMEM((HALF, D), jnp.float32),
            pltpu.SemaphoreType.DMA,
            pltpu.SemaphoreType.DMA,
            pltpu.SemaphoreType.DMA,
            pltpu.SemaphoreType.DMA,
        ],
    )


def _mlp_body(ue_ref, ie_ref, w1u_ref, w1i_ref, b1_ref, w2_ref, b2_ref,
              w3t_ref, b3_ref, out_ref):
    h = jnp.dot(ue_ref[...], w1u_ref[...], preferred_element_type=jnp.float32)
    h += jnp.dot(ie_ref[...], w1i_ref[...], preferred_element_type=jnp.float32)
    h = jnp.maximum(h + b1_ref[...], 0.0)
    h = jnp.dot(h, w2_ref[...], preferred_element_type=jnp.float32)
    h = jnp.maximum(h + b2_ref[...], 0.0)
    o = jax.lax.dot_general(w3t_ref[...], h, (((1,), (1,)), ((), ())),
                            preferred_element_type=jnp.float32)
    out_ref[...] = jax.nn.sigmoid(o + b3_ref[...])[None]


_BB = 4096


def _mlp(ue, ie, w1u, w1i, b1, w2, b2, w3t, b3):
    return pl.pallas_call(
        _mlp_body,
        grid=(CH // _BB,),
        in_specs=[
            pl.BlockSpec((_BB, D), lambda i: (i, 0)),
            pl.BlockSpec((_BB, D), lambda i: (i, 0)),
            pl.BlockSpec((D, 64), lambda i: (0, 0)),
            pl.BlockSpec((D, 64), lambda i: (0, 0)),
            pl.BlockSpec((1, 64), lambda i: (0, 0)),
            pl.BlockSpec((64, 32), lambda i: (0, 0)),
            pl.BlockSpec((1, 32), lambda i: (0, 0)),
            pl.BlockSpec((1, 32), lambda i: (0, 0)),
            pl.BlockSpec((1, 1), lambda i: (0, 0)),
        ],
        out_specs=pl.BlockSpec((1, 1, _BB), lambda i: (i, 0, 0)),
        out_shape=jax.ShapeDtypeStruct((CH // _BB, 1, _BB), jnp.float32),
    )(ue, ie, w1u, w1i, b1, w2, b2, w3t, b3)


@jax.jit
def kernel(user, item, user_table, item_table, W1, b1, W2, b2, W3, b3):
    w1u, w1i = W1[:D], W1[D:]
    b1r = b1.reshape(1, 64)
    b2r = b2.reshape(1, 32)
    w3t = W3.reshape(1, 32)
    b3r = b3.reshape(1, 1)
    gather = _gather_kernel()
    outs = []
    for c in range(NCH):
        ue, ie = gather(user_table, item_table,
                        user[c * CH:(c + 1) * CH],
                        item[c * CH:(c + 1) * CH])
        outs.append(_mlp(ue, ie, w1u, w1i, b1r, W2, b2r, w3t, b3r))
    return jnp.concatenate(outs, axis=0).reshape(B)


# speedup vs baseline: 1.1068x; 1.0045x over previous
import functools

import jax
import jax.numpy as jnp
from jax import lax
from jax.experimental import pallas as pl
from jax.experimental.pallas import tpu as pltpu
from jax.experimental.pallas import tpu_sc as plsc

B = 16384
D = 128
NC, NS = 2, 16
NW = NC * NS
NCH = 1
CH = B // NCH
B_PER_W = CH // NW
HALF = B_PER_W // 2


def _gather_body(user_tab, item_tab, uidx_hbm, iidx_hbm, ue_hbm, ie_hbm,
                 uidx_v, iidx_v, r0, r1, g0, g1, w0, w1):
    wid = lax.axis_index("s") * NC + lax.axis_index("c")
    base = wid * B_PER_W
    pltpu.sync_copy(uidx_hbm.at[pl.ds(base, B_PER_W)], uidx_v)
    pltpu.sync_copy(iidx_hbm.at[pl.ds(base, B_PER_W)], iidx_v)

    items = [
        (uidx_v, user_tab, ue_hbm, 0),
        (uidx_v, user_tab, ue_hbm, HALF),
        (iidx_v, item_tab, ie_hbm, 0),
        (iidx_v, item_tab, ie_hbm, HALF),
    ]
    bufs = (r0, r1)
    gsems = (g0, g1)
    wsems = (w0, w1)

    copies = [None, None, None, None]
    writes = [None, None]
    for k, (idx_v, tab, out_hbm, off) in enumerate(items):
        b = k % 2
        if writes[b] is not None:
            writes[b].wait()
        copies[k] = pltpu.async_copy(
            tab.at[idx_v.at[pl.ds(off, HALF)]], bufs[b], gsems[b])
        if k >= 1:
            copies[k - 1].wait()
            pk = k - 1
            pidx, ptab, pout, poff = items[pk]
            writes[pk % 2] = pltpu.async_copy(
                bufs[pk % 2], pout.at[pl.ds(base + poff, HALF)], wsems[pk % 2])
    copies[3].wait()
    writes[0].wait()
    writes[1] = pltpu.async_copy(bufs[1], ie_hbm.at[pl.ds(base + HALF, HALF)],
                                 wsems[1])
    writes[1].wait()


@functools.lru_cache(maxsize=1)
def _gather_kernel():
    mesh = plsc.VectorSubcoreMesh(core_axis_name="c", subcore_axis_name="s",
                                  num_cores=NC, num_subcores=NS)
    return pl.kernel(
        _gather_body,
        out_type=[
            jax.ShapeDtypeStruct((CH, D), jnp.float32),
            jax.ShapeDtypeStruct((CH, D), jnp.float32),
        ],
        mesh=mesh,
        scratch_types=[
            pltpu.VMEM((B_PER_W,), jnp.int32),
            pltpu.VMEM((B_PER_W,), jnp.int32),
            pltpu.VMEM((HALF, D), jnp.float32),
            pltpu.VMEM((HALF, D), jnp.float32),
            pltpu.SemaphoreType.DMA,
            pltpu.SemaphoreType.DMA,
            pltpu.SemaphoreType.DMA,
            pltpu.SemaphoreType.DMA,
        ],
    )


def _mlp_body(ue_ref, ie_ref, w1u_ref, w1i_ref, b1_ref, w2_ref, b2_ref,
              w3t_ref, b3_ref, out_ref):
    h = jnp.dot(ue_ref[...], w1u_ref[...], preferred_element_type=jnp.float32)
    h += jnp.dot(ie_ref[...], w1i_ref[...], preferred_element_type=jnp.float32)
    h = jnp.maximum(h + b1_ref[...], 0.0)
    h = jnp.dot(h, w2_ref[...], preferred_element_type=jnp.float32)
    h = jnp.maximum(h + b2_ref[...], 0.0)
    o = jax.lax.dot_general(w3t_ref[...], h, (((1,), (1,)), ((), ())),
                            preferred_element_type=jnp.float32)
    out_ref[...] = jax.nn.sigmoid(o + b3_ref[...])[None]


_BB = 8192


def _mlp(ue, ie, w1u, w1i, b1, w2, b2, w3t, b3):
    return pl.pallas_call(
        _mlp_body,
        grid=(CH // _BB,),
        in_specs=[
            pl.BlockSpec((_BB, D), lambda i: (i, 0)),
            pl.BlockSpec((_BB, D), lambda i: (i, 0)),
            pl.BlockSpec((D, 64), lambda i: (0, 0)),
            pl.BlockSpec((D, 64), lambda i: (0, 0)),
            pl.BlockSpec((1, 64), lambda i: (0, 0)),
            pl.BlockSpec((64, 32), lambda i: (0, 0)),
            pl.BlockSpec((1, 32), lambda i: (0, 0)),
            pl.BlockSpec((1, 32), lambda i: (0, 0)),
            pl.BlockSpec((1, 1), lambda i: (0, 0)),
        ],
        out_specs=pl.BlockSpec((1, 1, _BB), lambda i: (i, 0, 0)),
        out_shape=jax.ShapeDtypeStruct((CH // _BB, 1, _BB), jnp.float32),
    )(ue, ie, w1u, w1i, b1, w2, b2, w3t, b3)


@jax.jit
def kernel(user, item, user_table, item_table, W1, b1, W2, b2, W3, b3):
    w1u, w1i = W1[:D], W1[D:]
    b1r = b1.reshape(1, 64)
    b2r = b2.reshape(1, 32)
    w3t = W3.reshape(1, 32)
    b3r = b3.reshape(1, 1)
    gather = _gather_kernel()
    outs = []
    for c in range(NCH):
        ue, ie = gather(user_table, item_table,
                        user[c * CH:(c + 1) * CH],
                        item[c * CH:(c + 1) * CH])
        outs.append(_mlp(ue, ie, w1u, w1i, b1r, W2, b2r, w3t, b3r))
    return jnp.concatenate(outs, axis=0).reshape(B)
